# baseline (speedup 1.0000x reference)
import dataclasses
import functools

import jax
import jax.numpy as jnp
from jax import lax
from jax.experimental import pallas as pl
from jax.experimental.pallas import tpu as pltpu
from jax.experimental.pallas import tpu_sc as plsc

_L = 16
_COLS = 128
_FSEG = 25
_NSEG = 4
_NACC = _L + 1


@functools.partial(jax.jit, static_argnums=(1, 2, 3))
def _sc_pool_t(xt, b, f, d):
    mesh = plsc.VectorSubcoreMesh(core_axis_name="core", subcore_axis_name="subcore")
    cp = pltpu.CompilerParams()
    if "needs_layout_passes" in pltpu.CompilerParams.__dataclass_fields__:
        cp = dataclasses.replace(cp, needs_layout_passes=False)
    seg_rows = _FSEG * d
    n_lg = _COLS // _L

    @functools.partial(
        pl.kernel,
        out_type=jax.ShapeDtypeStruct((b,), jnp.float32),
        mesh=mesh,
        compiler_params=cp,
        scratch_types=[pltpu.VMEM((n_lg * _NACC * _L,), jnp.float32)],
    )
    def k(x_hbm, o_hbm, acc_ref):
        def body(x_vmem, o_vmem, acc):
            r = pl.program_id(1)
            first = r == 0

            @pl.loop(0, n_lg)
            def per_lane_group(g):
                base = g * (_NACC * _L)
                ss = [
                    jnp.where(first, 0.0, acc[pl.ds(base + t * _L, _L)])
                    for t in range(_L)
                ]
                qs = [jnp.where(first, 0.0, acc[pl.ds(base + _L * _L, _L)])]
                qs += [jnp.zeros((_L,), jnp.float32) for _ in range(3)]
                for fi in range(_FSEG):
                    for dd in range(d):
                        v = x_vmem[fi * d + dd, pl.ds(g * _L, _L)]
                        ss[dd] = ss[dd] + v
                        c = dd % 4
                        qs[c] = qs[c] + v * v
                q = (qs[0] + qs[1]) + (qs[2] + qs[3])
                for t in range(_L):
                    acc[pl.ds(base + t * _L, _L)] = ss[t]
                acc[pl.ds(base + _L * _L, _L)] = q
                tot = ss[0] * ss[0]
                for t in range(1, _L):
                    tot = tot + ss[t] * ss[t]
                o_vmem[pl.ds(g * _L, _L)] = (tot - q) * 0.5

        pltpu.emit_pipeline(
            body,
            grid=(b // _COLS, _NSEG),
            in_specs=[pl.BlockSpec((seg_rows, _COLS), lambda i, j: (j, i))],
            out_specs=[pl.BlockSpec((_COLS,), lambda i, j: (i,))],
            core_axis_name=("core", "subcore"),
            dimension_semantics=(pltpu.PARALLEL, pltpu.ARBITRARY),
        )(x_hbm, o_hbm, scratches=[acc_ref])

    return k(xt)


def kernel(feature_emb):
    b, f, d = feature_emb.shape
    xt = feature_emb.transpose(1, 2, 0).reshape(f * d, b)
    return _sc_pool_t(xt, b, f, d).reshape(b, 1)


# speedup vs baseline: 3.1965x; 3.1965x over previous
import dataclasses
import functools

import jax
import jax.numpy as jnp
from jax import lax
from jax.experimental import pallas as pl
from jax.experimental.pallas import tpu as pltpu
from jax.experimental.pallas import tpu_sc as plsc

_L = 16
_COLS = 128
_FSEG = 25
_NSEG = 4
_NACC = _L + 1


@functools.partial(jax.jit, static_argnums=(1, 2, 3))
def _sc_pool_t(xt, b, f, d):
    mesh = plsc.VectorSubcoreMesh(core_axis_name="core", subcore_axis_name="subcore")
    cp = pltpu.CompilerParams()
    if "needs_layout_passes" in pltpu.CompilerParams.__dataclass_fields__:
        cp = dataclasses.replace(cp, needs_layout_passes=False)
    seg_rows = _FSEG * d
    n_lg = _COLS // _L

    @functools.partial(
        pl.kernel,
        out_type=jax.ShapeDtypeStruct((b,), jnp.float32),
        mesh=mesh,
        compiler_params=cp,
        scratch_types=[pltpu.VMEM((n_lg * _NACC * _L,), jnp.float32)],
    )
    def k(x_hbm, o_hbm, acc_ref):
        def body(x_vmem, o_vmem, acc):
            r = pl.program_id(1)
            first = r == 0

            @pl.loop(0, n_lg)
            def per_lane_group(g):
                base = g * (_NACC * _L)
                ss = [
                    jnp.where(first, 0.0, acc[pl.ds(base + t * _L, _L)])
                    for t in range(_L)
                ]
                qs = (
                    jnp.where(first, 0.0, acc[pl.ds(base + _L * _L, _L)]),
                    jnp.zeros((_L,), jnp.float32),
                    jnp.zeros((_L,), jnp.float32),
                    jnp.zeros((_L,), jnp.float32),
                )

                def fstep(fi, carry):
                    row = fi * d
                    s = list(carry[:_L])
                    q = list(carry[_L:])
                    for dd in range(d):
                        v = x_vmem[row + dd, pl.ds(g * _L, _L)]
                        s[dd] = s[dd] + v
                        q[dd % 4] = q[dd % 4] + v * v
                    return tuple(s) + tuple(q)

                state = lax.fori_loop(0, _FSEG, fstep, tuple(ss) + qs)
                ss = state[:_L]
                q = (state[_L] + state[_L + 1]) + (state[_L + 2] + state[_L + 3])
                for t in range(_L):
                    acc[pl.ds(base + t * _L, _L)] = ss[t]
                acc[pl.ds(base + _L * _L, _L)] = q
                tot = ss[0] * ss[0]
                for t in range(1, _L):
                    tot = tot + ss[t] * ss[t]
                o_vmem[pl.ds(g * _L, _L)] = (tot - q) * 0.5

        pltpu.emit_pipeline(
            body,
            grid=(b // _COLS, _NSEG),
            in_specs=[pl.BlockSpec((seg_rows, _COLS), lambda i, j: (j, i))],
            out_specs=[pl.BlockSpec((_COLS,), lambda i, j: (i,))],
            core_axis_name=("core", "subcore"),
            dimension_semantics=(pltpu.PARALLEL, pltpu.ARBITRARY),
        )(x_hbm, o_hbm, scratches=[acc_ref])

    return k(xt)


def kernel(feature_emb):
    b, f, d = feature_emb.shape
    xt = feature_emb.transpose(1, 2, 0).reshape(f * d, b)
    return _sc_pool_t(xt, b, f, d).reshape(b, 1)
